# Initial kernel scaffold; baseline (speedup 1.0000x reference)
import functools
import math

import jax
import jax.numpy as jnp
from jax import lax
from jax.experimental import pallas as pl
from jax.experimental.pallas import tpu as pltpu
from jax.experimental.pallas import tpu_sc as plsc

N_ATOMS = 10000
N_EDGES = 320000
D = 128
D_RBF = 16
INV_SQRT_2 = 1.0 / math.sqrt(2.0)


_BG = 4000


def _stage1_body(m_ref, b_ref, w_ref, x_ref):
    emb = jnp.dot(b_ref[...], w_ref[...], preferred_element_type=jnp.float32)
    x_ref[...] = m_ref[...] * emb


def _stage1(m, basis, w_eff):
    grid = N_EDGES // _BG
    return pl.pallas_call(
        _stage1_body,
        grid=(grid,),
        in_specs=[
            pl.BlockSpec((_BG, D), lambda g: (g, 0)),
            pl.BlockSpec((_BG, D_RBF), lambda g: (g, 0)),
            pl.BlockSpec((D_RBF, D), lambda g: (0, 0)),
        ],
        out_specs=pl.BlockSpec((_BG, D), lambda g: (g, 0)),
        out_shape=jax.ShapeDtypeStruct((N_EDGES, D), jnp.float32),
    )(m, basis, w_eff)


_C = 128
_NCHUNKS = N_EDGES // _C
_NW = 32
_ROWS_PER_SUB = N_ATOMS // 16


def _sc_scatter_body(x_hbm, idx_hbm, zeros_hbm, out_hbm, xv, idxv, acc):
    c = lax.axis_index("c")
    s = lax.axis_index("s")
    wid = c * 16 + s

    pltpu.sync_copy(zeros_hbm.at[pl.ds(0, _ROWS_PER_SUB)],
                    acc.at[pl.ds(s * _ROWS_PER_SUB, _ROWS_PER_SUB)])
    plsc.subcore_barrier()

    n_it = jnp.where(wid < (_NCHUNKS % _NW), _NCHUNKS // _NW + 1, _NCHUNKS // _NW)

    def body(it, carry):
        base = (wid + it * _NW) * _C
        pltpu.sync_copy(x_hbm.at[pl.ds(base, _C)], xv)
        pltpu.sync_copy(idx_hbm.at[pl.ds(base, _C)], idxv)
        pltpu.sync_copy(xv, acc.at[idxv], add=True)
        return carry

    lax.fori_loop(0, n_it, body, 0)
    plsc.subcore_barrier()

    pltpu.sync_copy(acc.at[pl.ds(s * _ROWS_PER_SUB, _ROWS_PER_SUB)],
                    out_hbm.at[c].at[pl.ds(s * _ROWS_PER_SUB, _ROWS_PER_SUB)])


def _stage2(x, idx, zeros_rows):
    mesh = plsc.VectorSubcoreMesh(core_axis_name="c", subcore_axis_name="s")
    f = pl.kernel(
        _sc_scatter_body,
        out_type=jax.ShapeDtypeStruct((2, N_ATOMS, D), jnp.float32),
        mesh=mesh,
        scratch_types=[
            pltpu.VMEM((_C, D), jnp.float32),
            pltpu.VMEM((_C,), jnp.int32),
            pltpu.VMEM_SHARED((N_ATOMS, D), jnp.float32),
        ],
    )
    return f(x, idx, zeros_rows)


_BA = 1000


def _ssilu(x):
    sig = 1.0 / (1.0 + jnp.exp(-x))
    return x * sig * (1.0 / 0.6)


def _stage3_body(p_ref, wa0_ref, wb0_ref, wa1_ref, wb1_ref, o_ref):
    x = p_ref[0] + p_ref[1]
    for wa, wb in ((wa0_ref, wb0_ref), (wa1_ref, wb1_ref)):
        y = _ssilu(jnp.dot(x, wa[...], preferred_element_type=jnp.float32))
        y = _ssilu(jnp.dot(y, wb[...], preferred_element_type=jnp.float32))
        x = (x + y) * INV_SQRT_2
    o_ref[...] = x


def _stage3(partials, wa0, wb0, wa1, wb1):
    grid = N_ATOMS // _BA
    wspec = pl.BlockSpec((D, D), lambda g: (0, 0))
    return pl.pallas_call(
        _stage3_body,
        grid=(grid,),
        in_specs=[
            pl.BlockSpec((2, _BA, D), lambda g: (0, g, 0)),
            wspec, wspec, wspec, wspec,
        ],
        out_specs=pl.BlockSpec((_BA, D), lambda g: (g, 0)),
        out_shape=jax.ShapeDtypeStruct((N_ATOMS, D), jnp.float32),
    )(partials, wa0, wb0, wa1, wb1)


def kernel(h, m, basis_rad, idx_atom, W_rbf, scale_sum, W_r0a, W_r0b, W_r1a, W_r1b):
    del h
    w_eff = W_rbf * scale_sum
    x = _stage1(m, basis_rad, w_eff)
    zeros_rows = jnp.zeros((_ROWS_PER_SUB, D), jnp.float32)
    partials = _stage2(x, idx_atom, zeros_rows)
    return _stage3(partials, W_r0a, W_r0b, W_r1a, W_r1b)


# speedup vs baseline: 2.4367x; 2.4367x over previous
import functools
import math

import jax
import jax.numpy as jnp
from jax import lax
from jax.experimental import pallas as pl
from jax.experimental.pallas import tpu as pltpu
from jax.experimental.pallas import tpu_sc as plsc

N_ATOMS = 10000
N_EDGES = 320000
D = 128
D_RBF = 16
INV_SQRT_2 = 1.0 / math.sqrt(2.0)


_BG = 4000


def _stage1_body(m_ref, b_ref, w_ref, x_ref):
    emb = jnp.dot(b_ref[...], w_ref[...], preferred_element_type=jnp.float32)
    x_ref[...] = m_ref[...] * emb


def _stage1(m, basis, w_eff):
    grid = N_EDGES // _BG
    return pl.pallas_call(
        _stage1_body,
        grid=(grid,),
        in_specs=[
            pl.BlockSpec((_BG, D), lambda g: (g, 0)),
            pl.BlockSpec((_BG, D_RBF), lambda g: (g, 0)),
            pl.BlockSpec((D_RBF, D), lambda g: (0, 0)),
        ],
        out_specs=pl.BlockSpec((_BG, D), lambda g: (g, 0)),
        out_shape=jax.ShapeDtypeStruct((N_EDGES, D), jnp.float32),
    )(m, basis, w_eff)


_C = 128
_NCHUNKS = N_EDGES // _C
_NW = 32
_N_PAD = 10240
_ROWS_PER_SUB = _N_PAD // 16


def _sc_scatter_body(x_hbm, idx_hbm, zeros_hbm, out_hbm, xv, idxv, acc):
    c = lax.axis_index("c")
    s = lax.axis_index("s")
    wid = c * 16 + s

    pltpu.sync_copy(zeros_hbm.at[pl.ds(0, _ROWS_PER_SUB)],
                    acc.at[pl.ds(s * _ROWS_PER_SUB, _ROWS_PER_SUB)])
    plsc.subcore_barrier()

    n_it = jnp.where(wid < (_NCHUNKS % _NW), _NCHUNKS // _NW + 1, _NCHUNKS // _NW)

    def body(it, carry):
        base = (wid + it * _NW) * _C
        pltpu.sync_copy(x_hbm.at[pl.ds(base, _C)], xv)
        pltpu.sync_copy(idx_hbm.at[pl.ds(base, _C)], idxv)
        pltpu.sync_copy(xv, acc.at[idxv], add=True)
        return carry

    lax.fori_loop(0, n_it, body, 0)
    plsc.subcore_barrier()

    pltpu.sync_copy(acc.at[pl.ds(s * _ROWS_PER_SUB, _ROWS_PER_SUB)],
                    out_hbm.at[c].at[pl.ds(s * _ROWS_PER_SUB, _ROWS_PER_SUB)])


def _stage2(x, idx, zeros_rows):
    mesh = plsc.VectorSubcoreMesh(core_axis_name="c", subcore_axis_name="s")
    f = pl.kernel(
        _sc_scatter_body,
        out_type=jax.ShapeDtypeStruct((2, _N_PAD, D), jnp.float32),
        mesh=mesh,
        scratch_types=[
            pltpu.VMEM((_C, D), jnp.float32),
            pltpu.VMEM((_C,), jnp.int32),
            pltpu.VMEM_SHARED((_N_PAD, D), jnp.float32),
        ],
    )
    return f(x, idx, zeros_rows)


_BA = 1000


def _ssilu(x):
    sig = 1.0 / (1.0 + jnp.exp(-x))
    return x * sig * (1.0 / 0.6)


def _stage3_body(p_ref, wa0_ref, wb0_ref, wa1_ref, wb1_ref, o_ref):
    x = p_ref[0] + p_ref[1]
    for wa, wb in ((wa0_ref, wb0_ref), (wa1_ref, wb1_ref)):
        y = _ssilu(jnp.dot(x, wa[...], preferred_element_type=jnp.float32))
        y = _ssilu(jnp.dot(y, wb[...], preferred_element_type=jnp.float32))
        x = (x + y) * INV_SQRT_2
    o_ref[...] = x


def _stage3(partials, wa0, wb0, wa1, wb1):
    grid = N_ATOMS // _BA
    wspec = pl.BlockSpec((D, D), lambda g: (0, 0))
    return pl.pallas_call(
        _stage3_body,
        grid=(grid,),
        in_specs=[
            pl.BlockSpec((2, _BA, D), lambda g: (0, g, 0)),
            wspec, wspec, wspec, wspec,
        ],
        out_specs=pl.BlockSpec((_BA, D), lambda g: (g, 0)),
        out_shape=jax.ShapeDtypeStruct((N_ATOMS, D), jnp.float32),
    )(partials, wa0, wb0, wa1, wb1)


def kernel(h, m, basis_rad, idx_atom, W_rbf, scale_sum, W_r0a, W_r0b, W_r1a, W_r1b):
    del h
    w_eff = W_rbf * scale_sum
    x = _stage1(m, basis_rad, w_eff)
    zeros_rows = jnp.zeros((_ROWS_PER_SUB, D), jnp.float32)
    partials = _stage2(x, idx_atom, zeros_rows)
    return _stage3(partials, W_r0a, W_r0b, W_r1a, W_r1b)


# speedup vs baseline: 4.4702x; 1.8345x over previous
import math

import jax
import jax.numpy as jnp
from jax import lax
from jax.experimental import pallas as pl
from jax.experimental.pallas import tpu as pltpu
from jax.experimental.pallas import tpu_sc as plsc

N_ATOMS = 10000
N_EDGES = 320000
D = 128
D_RBF = 16
INV_SQRT_2 = 1.0 / math.sqrt(2.0)


_BG = 6400


def _stage1_body(m_ref, bt_ref, w_ref, x_ref):
    emb = lax.dot_general(bt_ref[...], w_ref[...],
                          dimension_numbers=(((0,), (0,)), ((), ())),
                          preferred_element_type=jnp.float32)
    x_ref[...] = m_ref[...] * emb


def _stage1(m, basis_t, w_eff):
    grid = N_EDGES // _BG
    return pl.pallas_call(
        _stage1_body,
        grid=(grid,),
        in_specs=[
            pl.BlockSpec((_BG, D), lambda g: (g, 0)),
            pl.BlockSpec((D_RBF, _BG), lambda g: (0, g)),
            pl.BlockSpec((D_RBF, D), lambda g: (0, 0)),
        ],
        out_specs=pl.BlockSpec((_BG, D), lambda g: (g, 0)),
        out_shape=jax.ShapeDtypeStruct((N_EDGES, D), jnp.float32),
    )(m, basis_t, w_eff)


_C = 128
_NCHUNKS = N_EDGES // _C
_BF = 1
_BROWS = _BF * _C
_NBLK = _NCHUNKS // _BF
_NW = 32
_IDXROWS = 88
_IDXPAD_ROWS = 2512
_N_PAD = 10240
_ROWS_PER_SUB = _N_PAD // 16


def _sc_scatter_body(x_hbm, idx2_hbm, zeros_hbm, out_hbm, xb0, xb1, idxv, acc,
                     sem0, sem1):
    c = lax.axis_index("c")
    s = lax.axis_index("s")
    wid = c * 16 + s

    pltpu.sync_copy(zeros_hbm.at[pl.ds(0, _ROWS_PER_SUB)],
                    acc.at[pl.ds(s * _ROWS_PER_SUB, _ROWS_PER_SUB)])

    lo = (wid * _NBLK) // _NW
    hi = ((wid + 1) * _NBLK) // _NW
    t = hi - lo
    lo2 = lo * _BF
    a0 = pl.multiple_of((lo2 >> 3) << 3, 8)
    ishift = lo2 - a0
    pltpu.sync_copy(idx2_hbm.at[pl.ds(a0, _IDXROWS)], idxv)

    def start(buf, sem, i):
        off = pl.multiple_of((lo + i) * _BROWS, 8)
        pltpu.async_copy(x_hbm.at[pl.ds(off, _BROWS)], buf, sem)

    def wait(buf, sem, i):
        off = pl.multiple_of((lo + i) * _BROWS, 8)
        pltpu.make_async_copy(x_hbm.at[pl.ds(off, _BROWS)], buf, sem).wait()

    def scat(buf, i):
        for b in range(_BF):
            pltpu.sync_copy(buf.at[pl.ds(b * _C, _C)],
                            acc.at[idxv.at[ishift + i * _BF + b]], add=True)

    plsc.subcore_barrier()
    start(xb0, sem0, 0)

    def pair(p, carry):
        i0 = 2 * p
        i1 = i0 + 1

        @pl.when(i1 < t)
        def _():
            start(xb1, sem1, i1)

        wait(xb0, sem0, i0)
        scat(xb0, i0)

        @pl.when(i1 < t)
        def _():
            @pl.when(i1 + 1 < t)
            def _():
                start(xb0, sem0, i1 + 1)

            wait(xb1, sem1, i1)
            scat(xb1, i1)

        return carry

    lax.fori_loop(0, (t + 1) // 2, pair, 0)
    plsc.subcore_barrier()

    pltpu.sync_copy(acc.at[pl.ds(s * _ROWS_PER_SUB, _ROWS_PER_SUB)],
                    out_hbm.at[c].at[pl.ds(s * _ROWS_PER_SUB, _ROWS_PER_SUB)])


def _stage2(x, idx2, zeros_rows):
    mesh = plsc.VectorSubcoreMesh(core_axis_name="c", subcore_axis_name="s")
    f = pl.kernel(
        _sc_scatter_body,
        out_type=jax.ShapeDtypeStruct((2, _N_PAD, D), jnp.float32),
        mesh=mesh,
        scratch_types=[
            pltpu.VMEM((_BROWS, D), jnp.float32),
            pltpu.VMEM((_BROWS, D), jnp.float32),
            pltpu.VMEM((_IDXROWS, _C), jnp.int32),
            pltpu.VMEM_SHARED((_N_PAD, D), jnp.float32),
            pltpu.SemaphoreType.DMA,
            pltpu.SemaphoreType.DMA,
        ],
    )
    return f(x, idx2, zeros_rows)


_BA = 1000


def _ssilu(x):
    sig = 1.0 / (1.0 + jnp.exp(-x))
    return x * sig * (1.0 / 0.6)


def _stage3_body(p_ref, wa0_ref, wb0_ref, wa1_ref, wb1_ref, o_ref):
    x = p_ref[0] + p_ref[1]
    for wa, wb in ((wa0_ref, wb0_ref), (wa1_ref, wb1_ref)):
        y = _ssilu(jnp.dot(x, wa[...], preferred_element_type=jnp.float32))
        y = _ssilu(jnp.dot(y, wb[...], preferred_element_type=jnp.float32))
        x = (x + y) * INV_SQRT_2
    o_ref[...] = x


def _stage3(partials, wa0, wb0, wa1, wb1):
    grid = N_ATOMS // _BA
    wspec = pl.BlockSpec((D, D), lambda g: (0, 0))
    return pl.pallas_call(
        _stage3_body,
        grid=(grid,),
        in_specs=[
            pl.BlockSpec((2, _BA, D), lambda g: (0, g, 0)),
            wspec, wspec, wspec, wspec,
        ],
        out_specs=pl.BlockSpec((_BA, D), lambda g: (g, 0)),
        out_shape=jax.ShapeDtypeStruct((N_ATOMS, D), jnp.float32),
    )(partials, wa0, wb0, wa1, wb1)


def kernel(h, m, basis_rad, idx_atom, W_rbf, scale_sum, W_r0a, W_r0b, W_r1a, W_r1b):
    del h
    w_eff = W_rbf * scale_sum
    x = _stage1(m, basis_rad.T, w_eff)
    idx2 = jnp.concatenate(
        [idx_atom, jnp.zeros((_IDXPAD_ROWS * _C - N_EDGES,), jnp.int32)]
    ).reshape(_IDXPAD_ROWS, _C)
    zeros_rows = jnp.zeros((_ROWS_PER_SUB, D), jnp.float32)
    partials = _stage2(x, idx2, zeros_rows)
    return _stage3(partials, W_r0a, W_r0b, W_r1a, W_r1b)
